# Initial kernel scaffold; baseline (speedup 1.0000x reference)
import jax
import jax.numpy as jnp
from jax.experimental import pallas as pl


def kernel(x, gate_W, gate_b, W0, b0, W1, b1, W2, b2):
    raise NotImplementedError("write your pallas kernel here")


# speedup vs baseline: 1.8856x; 1.8856x over previous
import functools

import jax
import jax.numpy as jnp
from jax.experimental import pallas as pl
from jax.experimental.pallas import tpu as pltpu

E = 8
D = 768
OUT = 768


def _moe_dense_kernel(x_ref, gw_ref, gb_ref, w0_ref, b0_ref, w1_ref, b1_ref,
                      w2_ref, b2_ref, out_ref):
    e = pl.program_id(1)
    x = x_ref[...]

    xg = x[:, D - 3:]
    gates = jax.lax.dot_general(
        xg, gw_ref[...], (((1,), (1,)), ((), ())),
        preferred_element_type=jnp.float32) + gb_ref[...][None, :]

    idx = jax.lax.broadcasted_iota(jnp.int32, gates.shape, 1)
    v1 = jnp.max(gates, axis=-1, keepdims=True)
    i1 = jnp.min(jnp.where(gates == v1, idx, E), axis=-1, keepdims=True)
    masked = jnp.where(idx == i1, -jnp.inf, gates)
    v2 = jnp.max(masked, axis=-1, keepdims=True)
    i2 = jnp.min(jnp.where(masked == v2, idx, E), axis=-1, keepdims=True)
    t = jnp.exp(v2 - v1)
    w1v = 1.0 / (1.0 + t)
    w2v = t / (1.0 + t)
    w = jnp.where(i1 == e, w1v, 0.0) + jnp.where(i2 == e, w2v, 0.0)

    h = jax.lax.dot_general(x, w0_ref[0], (((1,), (1,)), ((), ())),
                            preferred_element_type=jnp.float32)
    h = jnp.maximum(h + b0_ref[0], 0.0)
    h = jax.lax.dot_general(h, w1_ref[0], (((1,), (1,)), ((), ())),
                            preferred_element_type=jnp.float32)
    h = jnp.maximum(h + b1_ref[0], 0.0)
    o = jax.lax.dot_general(h, w2_ref[0], (((1,), (1,)), ((), ())),
                            preferred_element_type=jnp.float32)
    o = (o + b2_ref[0]) * w

    @pl.when(e == 0)
    def _():
        out_ref[...] = o

    @pl.when(e != 0)
    def _():
        out_ref[...] += o


@functools.partial(jax.jit, static_argnames=("interpret",))
def _moe_dense(x_flat, gate_W, gate_b, W0, b0, W1, b1, W2, b2,
               interpret=False):
    N = x_flat.shape[0]
    M = 1024
    nb = N // M
    out = pl.pallas_call(
        _moe_dense_kernel,
        grid=(nb, E),
        in_specs=[
            pl.BlockSpec((M, D), lambda b, e: (b, 0)),
            pl.BlockSpec((E, 3), lambda b, e: (0, 0)),
            pl.BlockSpec((E,), lambda b, e: (0,)),
            pl.BlockSpec((1, D, D), lambda b, e: (e, 0, 0)),
            pl.BlockSpec((1, 1, D), lambda b, e: (e, 0, 0)),
            pl.BlockSpec((1, D, D), lambda b, e: (e, 0, 0)),
            pl.BlockSpec((1, 1, D), lambda b, e: (e, 0, 0)),
            pl.BlockSpec((1, OUT, D), lambda b, e: (e, 0, 0)),
            pl.BlockSpec((1, 1, OUT), lambda b, e: (e, 0, 0)),
        ],
        out_specs=pl.BlockSpec((M, OUT), lambda b, e: (b, 0)),
        out_shape=jax.ShapeDtypeStruct((N, OUT), x_flat.dtype),
        compiler_params=pltpu.CompilerParams(
            dimension_semantics=("arbitrary", "arbitrary")),
        interpret=interpret,
    )(x_flat, gate_W, gate_b, W0, b0.reshape(E, 1, D), W1,
      b1.reshape(E, 1, D), W2, b2.reshape(E, 1, OUT))
    return out


def kernel(x, gate_W, gate_b, W0, b0, W1, b1, W2, b2):
    bsz, num_pairs, feat = x.shape
    x_flat = x.reshape(-1, feat)
    out = _moe_dense(x_flat, gate_W, gate_b, W0, b0, W1, b1, W2, b2)
    return out.reshape(bsz, num_pairs, OUT)


# speedup vs baseline: 1.9195x; 1.0180x over previous
import functools

import jax
import jax.numpy as jnp
from jax.experimental import pallas as pl
from jax.experimental.pallas import tpu as pltpu

E = 8
D = 768
OUT = 768


def _moe_dense_kernel(x_ref, gw_ref, gb_ref, w0_ref, b0_ref, w1_ref, b1_ref,
                      w2_ref, b2_ref, out_ref):
    e = pl.program_id(1)
    x = x_ref[...]

    xg = x[:, D - 3:]
    gates = jax.lax.dot_general(
        xg, gw_ref[...], (((1,), (1,)), ((), ())),
        preferred_element_type=jnp.float32) + gb_ref[...][None, :]

    idx = jax.lax.broadcasted_iota(jnp.int32, gates.shape, 1)
    v1 = jnp.max(gates, axis=-1, keepdims=True)
    i1 = jnp.min(jnp.where(gates == v1, idx, E), axis=-1, keepdims=True)
    masked = jnp.where(idx == i1, -jnp.inf, gates)
    v2 = jnp.max(masked, axis=-1, keepdims=True)
    i2 = jnp.min(jnp.where(masked == v2, idx, E), axis=-1, keepdims=True)
    t = jnp.exp(v2 - v1)
    w1v = 1.0 / (1.0 + t)
    w2v = t / (1.0 + t)
    w = jnp.where(i1 == e, w1v, 0.0) + jnp.where(i2 == e, w2v, 0.0)

    xb = x.astype(jnp.bfloat16)
    h = jax.lax.dot_general(xb, w0_ref[0].astype(jnp.bfloat16),
                            (((1,), (1,)), ((), ())),
                            preferred_element_type=jnp.float32)
    h = jnp.maximum(h + b0_ref[0], 0.0).astype(jnp.bfloat16)
    h = jax.lax.dot_general(h, w1_ref[0].astype(jnp.bfloat16),
                            (((1,), (1,)), ((), ())),
                            preferred_element_type=jnp.float32)
    h = jnp.maximum(h + b1_ref[0], 0.0).astype(jnp.bfloat16)
    o = jax.lax.dot_general(h, w2_ref[0].astype(jnp.bfloat16),
                            (((1,), (1,)), ((), ())),
                            preferred_element_type=jnp.float32)
    o = (o + b2_ref[0]) * w

    @pl.when(e == 0)
    def _():
        out_ref[...] = o

    @pl.when(e != 0)
    def _():
        out_ref[...] += o


@functools.partial(jax.jit, static_argnames=("interpret",))
def _moe_dense(x_flat, gate_W, gate_b, W0, b0, W1, b1, W2, b2,
               interpret=False):
    N = x_flat.shape[0]
    M = 2048
    nb = N // M
    out = pl.pallas_call(
        _moe_dense_kernel,
        grid=(nb, E),
        in_specs=[
            pl.BlockSpec((M, D), lambda b, e: (b, 0)),
            pl.BlockSpec((E, 3), lambda b, e: (0, 0)),
            pl.BlockSpec((E,), lambda b, e: (0,)),
            pl.BlockSpec((1, D, D), lambda b, e: (e, 0, 0)),
            pl.BlockSpec((1, 1, D), lambda b, e: (e, 0, 0)),
            pl.BlockSpec((1, D, D), lambda b, e: (e, 0, 0)),
            pl.BlockSpec((1, 1, D), lambda b, e: (e, 0, 0)),
            pl.BlockSpec((1, OUT, D), lambda b, e: (e, 0, 0)),
            pl.BlockSpec((1, 1, OUT), lambda b, e: (e, 0, 0)),
        ],
        out_specs=pl.BlockSpec((M, OUT), lambda b, e: (b, 0)),
        out_shape=jax.ShapeDtypeStruct((N, OUT), x_flat.dtype),
        compiler_params=pltpu.CompilerParams(
            dimension_semantics=("arbitrary", "arbitrary")),
        interpret=interpret,
    )(x_flat, gate_W, gate_b, W0, b0.reshape(E, 1, D), W1,
      b1.reshape(E, 1, D), W2, b2.reshape(E, 1, OUT))
    return out


def kernel(x, gate_W, gate_b, W0, b0, W1, b1, W2, b2):
    bsz, num_pairs, feat = x.shape
    x_flat = x.reshape(-1, feat)
    out = _moe_dense(x_flat, gate_W, gate_b, W0, b0, W1, b1, W2, b2)
    return out.reshape(bsz, num_pairs, OUT)
